# baseline (speedup 1.0000x reference)
import functools

import jax
import jax.numpy as jnp
from jax import lax
from jax.experimental import pallas as pl
from jax.experimental.pallas import tpu as pltpu
from jax.experimental.pallas import tpu_sc as plsc

_F32 = jnp.float32

_N = 10000
_E = 160000
_D = 248
_NNZ = 128
_L = 2
_DP = 256
_WU = _DP + _NNZ
_NC = 2
_NS = 16
_NW = _NC * _NS
_C = 16
_STEPS = 320
_EPW = _STEPS * _C
_EPAD = _EPW * _NW
_NP = 10240
_RPS = _NP // _NS


def _mlp_in_body(x_ref, w1_ref, b1_ref, w2_ref, b2_ref, o_ref):
    t = jnp.dot(x_ref[...], w1_ref[...], preferred_element_type=_F32) + b1_ref[...]
    t = t * jax.nn.sigmoid(t)
    o_ref[...] = jnp.dot(t, w2_ref[...], preferred_element_type=_F32) + b2_ref[...]


def _mlp_in(x, W1, b1, W2, b2):
    n, din = x.shape
    hid = W1.shape[1]
    d = W2.shape[1]
    blk = 1000
    return pl.pallas_call(
        _mlp_in_body,
        grid=(n // blk,),
        in_specs=[
            pl.BlockSpec((blk, din), lambda i: (i, 0)),
            pl.BlockSpec((din, hid), lambda i: (0, 0)),
            pl.BlockSpec((1, hid), lambda i: (0, 0)),
            pl.BlockSpec((hid, d), lambda i: (0, 0)),
            pl.BlockSpec((1, d), lambda i: (0, 0)),
        ],
        out_specs=pl.BlockSpec((blk, d), lambda i: (i, 0)),
        out_shape=jax.ShapeDtypeStruct((n, d), _F32),
    )(x, W1, b1, W2, b2)


def _uv_body(h_ref, mu_ref, u_ref, vi_ref):
    hv = h_ref[...]
    u_ref[...] = jnp.dot(hv, mu_ref[...], preferred_element_type=_F32)
    pad = jnp.zeros((hv.shape[0], _DP - _D), _F32)
    vi_ref[...] = jnp.concatenate([hv, pad], axis=1)


def _uv(h, MU):
    n, d = h.shape
    blk = 1000
    return pl.pallas_call(
        _uv_body,
        grid=(n // blk,),
        in_specs=[
            pl.BlockSpec((blk, d), lambda i: (i, 0)),
            pl.BlockSpec((d, _WU), lambda i: (0, 0)),
        ],
        out_specs=[
            pl.BlockSpec((blk, _WU), lambda i: (i, 0)),
            pl.BlockSpec((blk, _DP), lambda i: (i, 0)),
        ],
        out_shape=[
            jax.ShapeDtypeStruct((n, _WU), _F32),
            jax.ShapeDtypeStruct((n, _DP), _F32),
        ],
    )(h, MU)


def _update_body(h_ref, a0_ref, a1_ref, sj_ref, sk_ref, o_ref):
    hv = h_ref[...]
    vg = jnp.dot(hv, sj_ref[...], preferred_element_type=_F32)
    a = (a0_ref[...] + a1_ref[...]) * vg
    o_ref[...] = hv + jnp.dot(a, sk_ref[...], preferred_element_type=_F32)


def _update(h, a0, a1, Sj, Sk):
    n, d = h.shape
    nnz = Sk.shape[0]
    blk = 1000
    return pl.pallas_call(
        _update_body,
        grid=(n // blk,),
        in_specs=[
            pl.BlockSpec((blk, d), lambda i: (i, 0)),
            pl.BlockSpec((blk, nnz), lambda i: (i, 0)),
            pl.BlockSpec((blk, nnz), lambda i: (i, 0)),
            pl.BlockSpec((d, nnz), lambda i: (0, 0)),
            pl.BlockSpec((nnz, d), lambda i: (0, 0)),
        ],
        out_specs=pl.BlockSpec((blk, d), lambda i: (i, 0)),
        out_shape=jax.ShapeDtypeStruct((n, d), _F32),
    )(h, a0, a1, Sj, Sk)


def _readout_body(h_ref, k_ref, w1_ref, b1_ref, w2_ref, b2_ref, o_ref):
    hv = h_ref[...]
    hk = jnp.dot(hv, k_ref[...], preferred_element_type=_F32)
    s = jnp.sum(hk * hv, axis=1, keepdims=True)
    mh = jnp.mean(hv, axis=0, keepdims=True)
    ms = jnp.mean(s, axis=0, keepdims=True)
    pooled = jnp.concatenate([mh, ms], axis=1)
    t = jnp.dot(pooled, w1_ref[...], preferred_element_type=_F32) + b1_ref[...]
    t = t * jax.nn.sigmoid(t)
    o_ref[...] = jnp.dot(t, w2_ref[...], preferred_element_type=_F32) + b2_ref[...]


def _readout(h, K, Wo1, bo1, Wo2, bo2):
    out = Wo2.shape[1]
    return pl.pallas_call(
        _readout_body,
        out_shape=jax.ShapeDtypeStruct((1, out), _F32),
    )(h, K, Wo1, bo1, Wo2, bo2)


def _lane_sum(v):
    dnums = lax.GatherDimensionNumbers(
        offset_dims=(), collapsed_slice_dims=(0,), start_index_map=(0,))
    for shift in (8, 4, 2, 1):
        idx = jnp.bitwise_xor(lax.iota(jnp.int32, 16), shift)
        v = v + lax.gather(v, idx[:, None], dnums, (1,),
                           mode=lax.GatherScatterMode.PROMISE_IN_BOUNDS)
    return v


def _edge_body(u_hbm, vi_hbm, src_hbm, dst_hbm,
               g1w_hbm, g1b_hbm, g2w_hbm, g2b_hbm, z_hbm, out_hbm,
               idx_s, idx_d,
               bu0, bu1, bu2, bvi0, bvi1, bvi2,
               msg0, msg1, msg2, p1w, p1b, p2w, p2b, acc,
               sem_g0, sem_g1, sem_g2, sem_m0, sem_m1, sem_m2):
    core = lax.axis_index("c")
    sid = lax.axis_index("s")
    wid = sid * _NC + core
    bu = (bu0, bu1, bu2)
    bvi = (bvi0, bvi1, bvi2)
    msg = (msg0, msg1, msg2)
    sem_g = (sem_g0, sem_g1, sem_g2)
    sem_m = (sem_m0, sem_m1, sem_m2)

    def iv(ref, s):
        return ref[s // 8, pl.ds(pl.multiple_of((s % 8) * _C, _C), _C)]

    def fire(b, s):
        ivs = iv(idx_s, s)
        ivd = iv(idx_d, s)
        pltpu.async_copy(u_hbm.at[ivs], bu[b], sem_g[b])
        pltpu.async_copy(vi_hbm.at[ivd], bvi[b], sem_g[b])

    def wait_gathers(b, s):
        ivs = iv(idx_s, s)
        ivd = iv(idx_d, s)
        pltpu.make_async_copy(u_hbm.at[ivs], bu[b], sem_g[b]).wait()
        pltpu.make_async_copy(vi_hbm.at[ivd], bvi[b], sem_g[b]).wait()

    pltpu.sync_copy(g1w_hbm, p1w)
    pltpu.sync_copy(g1b_hbm, p1b)
    pltpu.sync_copy(g2w_hbm, p2w)
    pltpu.sync_copy(g2b_hbm, p2b)
    row0 = wid * (_STEPS // 8)
    pltpu.sync_copy(src_hbm.at[pl.ds(row0, _STEPS // 8)], idx_s)
    pltpu.sync_copy(dst_hbm.at[pl.ds(row0, _STEPS // 8)], idx_d)
    pltpu.sync_copy(z_hbm.at[pl.ds(sid * _RPS, _RPS)],
                    acc.at[pl.ds(sid * _RPS, _RPS)])
    plsc.subcore_barrier()

    def compute(b):
        ub, vi, mg = bu[b], bvi[b], msg[b]

        @plsc.parallel_loop(0, _C, unroll=2)
        def edge(e):
            accs = [jnp.zeros((16,), _F32) for _ in range(4)]
            for k in range(_DP // 16):
                u = ub[e, pl.ds(16 * k, 16)]
                v = vi[e, pl.ds(16 * k, 16)]
                accs[k % 4] = accs[k % 4] + u * v
            inv = _lane_sum(accs[0] + accs[1] + accs[2] + accs[3])
            ga = p2b[...]
            for k in range(_NNZ // 16):
                tt = inv * p1w[pl.ds(16 * k, 16)] + p1b[pl.ds(16 * k, 16)]
                sg = 1.0 / (1.0 + jnp.exp(-tt))
                ga = ga + (tt * sg) * p2w[pl.ds(16 * k, 16)]
            gate = 1.0 / (1.0 + jnp.exp(-_lane_sum(ga)))
            for k in range(_NNZ // 16):
                mg[e, pl.ds(16 * k, 16)] = gate * ub[e, pl.ds(_DP + 16 * k, 16)]

    def phase(b, s):
        wait_gathers(b, s)

        @pl.when(s >= 3)
        def _():
            pvd = iv(idx_d, s - 3)
            pltpu.make_async_copy(msg[b], acc.at[pvd], sem_m[b]).wait()

        compute(b)
        pltpu.async_copy(msg[b], acc.at[iv(idx_d, s)], sem_m[b], add=True)

        @pl.when(s + 3 < _STEPS)
        def _():
            fire(b, s + 3)

    for b in range(3):
        fire(b, b)

    def superstep(k, carry):
        for b in range(3):
            phase(b, 3 * k + b)
        return carry

    nfull = (_STEPS // 3) * 3
    lax.fori_loop(0, _STEPS // 3, superstep, 0)
    for s in range(nfull, _STEPS):
        phase(s % 3, jnp.int32(s))
    for s in range(_STEPS - 3, _STEPS):
        pltpu.make_async_copy(msg[s % 3], acc.at[iv(idx_d, s)],
                              sem_m[s % 3]).wait()
    plsc.subcore_barrier()
    pltpu.sync_copy(acc.at[pl.ds(sid * _RPS, _RPS)],
                    out_hbm.at[core, pl.ds(sid * _RPS, _RPS)])


def _edge_stage(U, Vi, src, dst, g1w, g1b, g2w, g2b16, zrows):
    call = pl.kernel(
        _edge_body,
        out_type=jax.ShapeDtypeStruct((_NC, _NP, _NNZ), _F32),
        mesh=plsc.VectorSubcoreMesh(core_axis_name="c", subcore_axis_name="s",
                                    num_cores=_NC, num_subcores=_NS),
        scratch_types=(
            [pltpu.VMEM((_STEPS // 8, 128), jnp.int32)] * 2
            + [pltpu.VMEM((_C, _WU), _F32)] * 3
            + [pltpu.VMEM((_C, _DP), _F32)] * 3
            + [pltpu.VMEM((_C, _NNZ), _F32)] * 3
            + [pltpu.VMEM((_NNZ,), _F32)] * 3
            + [pltpu.VMEM((16,), _F32)]
            + [pltpu.VMEM_SHARED((_NP, _NNZ), _F32)]
            + [pltpu.SemaphoreType.DMA] * 6
        ),
    )
    return call(U, Vi, src, dst, g1w, g1b, g2w, g2b16, zrows)


def kernel(x, edge_index, sc_i, sc_j, sc_k, sc_c, W_in1, b_in1, W_in2, b_in2,
           K_mat, g1_w, g1_b, g2_w, g2_b, Wo1, bo1, Wo2, bo2):
    npad = _EPAD - _E
    src = jnp.concatenate([edge_index[0], jnp.zeros((npad,), jnp.int32)])
    dst = jnp.concatenate([edge_index[1],
                           jnp.full((npad,), _NP - 1, jnp.int32)])
    src = src.reshape(-1, 128)
    dst = dst.reshape(-1, 128)
    dr = jnp.arange(_D, dtype=jnp.int32)
    Si = (dr[:, None] == sc_i[None, :]).astype(_F32) * sc_c[None, :]
    Sj = (dr[:, None] == sc_j[None, :]).astype(_F32)
    Sk = (sc_k[:, None] == dr[None, :]).astype(_F32)
    pad = jnp.zeros((_D, _DP - _D), _F32)
    MU = jnp.concatenate([K_mat, pad, Si], axis=1)
    zrows = jnp.zeros((_NP, _NNZ), _F32)

    h = _mlp_in(x, W_in1, b_in1.reshape(1, -1), W_in2, b_in2.reshape(1, -1))
    for l in range(_L):
        U, Vi = _uv(h, MU)
        g2b16 = jnp.full((16,), g2_b[l] / 16.0, _F32)
        acc = _edge_stage(U, Vi, src, dst,
                          g1_w[l], g1_b[l], g2_w[l], g2b16, zrows)
        h = _update(h, acc[0, :_N], acc[1, :_N], Sj, Sk)
    return _readout(h, K_mat, Wo1, bo1.reshape(1, -1), Wo2, bo2.reshape(1, -1))


# speedup vs baseline: 1.2129x; 1.2129x over previous
import functools

import jax
import jax.numpy as jnp
from jax import lax
from jax.experimental import pallas as pl
from jax.experimental.pallas import tpu as pltpu
from jax.experimental.pallas import tpu_sc as plsc

_F32 = jnp.float32

_N = 10000
_E = 160000
_D = 248
_NNZ = 128
_L = 2
_DP = 256
_WU = _DP + _NNZ
_NC = 2
_NS = 16
_NW = _NC * _NS
_C = 16
_STEPS = 320
_EPW = _STEPS * _C
_EPAD = _EPW * _NW
_NP = 10240
_RPS = _NP // _NS


def _mlp_in_body(x_ref, w1_ref, b1_ref, w2_ref, b2_ref, o_ref):
    t = jnp.dot(x_ref[...], w1_ref[...], preferred_element_type=_F32) + b1_ref[...]
    t = t * jax.nn.sigmoid(t)
    o_ref[...] = jnp.dot(t, w2_ref[...], preferred_element_type=_F32) + b2_ref[...]


def _mlp_in(x, W1, b1, W2, b2):
    n, din = x.shape
    hid = W1.shape[1]
    d = W2.shape[1]
    blk = 1000
    return pl.pallas_call(
        _mlp_in_body,
        grid=(n // blk,),
        in_specs=[
            pl.BlockSpec((blk, din), lambda i: (i, 0)),
            pl.BlockSpec((din, hid), lambda i: (0, 0)),
            pl.BlockSpec((1, hid), lambda i: (0, 0)),
            pl.BlockSpec((hid, d), lambda i: (0, 0)),
            pl.BlockSpec((1, d), lambda i: (0, 0)),
        ],
        out_specs=pl.BlockSpec((blk, d), lambda i: (i, 0)),
        out_shape=jax.ShapeDtypeStruct((n, d), _F32),
    )(x, W1, b1, W2, b2)


def _uv_body(h_ref, mui_ref, mug_ref, ui_ref, ug_ref, vi_ref):
    hv = h_ref[...]
    ui_ref[...] = jnp.dot(hv, mui_ref[...], preferred_element_type=_F32)
    ug_ref[...] = jnp.dot(hv, mug_ref[...], preferred_element_type=_F32)
    pad = jnp.zeros((hv.shape[0], _DP - _D), _F32)
    vi_ref[...] = jnp.concatenate([hv, pad], axis=1)


def _uv(h, MUi, MUg):
    n, d = h.shape
    blk = 1000
    return pl.pallas_call(
        _uv_body,
        grid=(n // blk,),
        in_specs=[
            pl.BlockSpec((blk, d), lambda i: (i, 0)),
            pl.BlockSpec((d, _DP), lambda i: (0, 0)),
            pl.BlockSpec((d, _NNZ), lambda i: (0, 0)),
        ],
        out_specs=[
            pl.BlockSpec((blk, _DP), lambda i: (i, 0)),
            pl.BlockSpec((blk, _NNZ), lambda i: (i, 0)),
            pl.BlockSpec((blk, _DP), lambda i: (i, 0)),
        ],
        out_shape=[
            jax.ShapeDtypeStruct((n, _DP), _F32),
            jax.ShapeDtypeStruct((n, _NNZ), _F32),
            jax.ShapeDtypeStruct((n, _DP), _F32),
        ],
    )(h, MUi, MUg)


def _update_body(h_ref, a0_ref, a1_ref, sj_ref, sk_ref, o_ref):
    hv = h_ref[...]
    vg = jnp.dot(hv, sj_ref[...], preferred_element_type=_F32)
    a = (a0_ref[...] + a1_ref[...]) * vg
    o_ref[...] = hv + jnp.dot(a, sk_ref[...], preferred_element_type=_F32)


def _update(h, a0, a1, Sj, Sk):
    n, d = h.shape
    nnz = Sk.shape[0]
    blk = 1000
    return pl.pallas_call(
        _update_body,
        grid=(n // blk,),
        in_specs=[
            pl.BlockSpec((blk, d), lambda i: (i, 0)),
            pl.BlockSpec((blk, nnz), lambda i: (i, 0)),
            pl.BlockSpec((blk, nnz), lambda i: (i, 0)),
            pl.BlockSpec((d, nnz), lambda i: (0, 0)),
            pl.BlockSpec((nnz, d), lambda i: (0, 0)),
        ],
        out_specs=pl.BlockSpec((blk, d), lambda i: (i, 0)),
        out_shape=jax.ShapeDtypeStruct((n, d), _F32),
    )(h, a0, a1, Sj, Sk)


def _readout_body(h_ref, k_ref, w1_ref, b1_ref, w2_ref, b2_ref, o_ref):
    hv = h_ref[...]
    hk = jnp.dot(hv, k_ref[...], preferred_element_type=_F32)
    s = jnp.sum(hk * hv, axis=1, keepdims=True)
    mh = jnp.mean(hv, axis=0, keepdims=True)
    ms = jnp.mean(s, axis=0, keepdims=True)
    pooled = jnp.concatenate([mh, ms], axis=1)
    t = jnp.dot(pooled, w1_ref[...], preferred_element_type=_F32) + b1_ref[...]
    t = t * jax.nn.sigmoid(t)
    o_ref[...] = jnp.dot(t, w2_ref[...], preferred_element_type=_F32) + b2_ref[...]


def _readout(h, K, Wo1, bo1, Wo2, bo2):
    out = Wo2.shape[1]
    return pl.pallas_call(
        _readout_body,
        out_shape=jax.ShapeDtypeStruct((1, out), _F32),
    )(h, K, Wo1, bo1, Wo2, bo2)


def _lane_sum(v):
    dnums = lax.GatherDimensionNumbers(
        offset_dims=(), collapsed_slice_dims=(0,), start_index_map=(0,))
    for shift in (8, 4, 2, 1):
        idx = jnp.bitwise_xor(lax.iota(jnp.int32, 16), shift)
        v = v + lax.gather(v, idx[:, None], dnums, (1,),
                           mode=lax.GatherScatterMode.PROMISE_IN_BOUNDS)
    return v


def _edge_body(ui_hbm, ug_hbm, vi_hbm, src_hbm, dst_hbm,
               g1w_hbm, g1b_hbm, g2w_hbm, g2b_hbm, z_hbm, out_hbm,
               idx_s, idx_d,
               bui0, bui1, bui2, bug0, bug1, bug2, bvi0, bvi1, bvi2,
               msg0, msg1, msg2, p1w, p1b, p2w, p2b, acc,
               sem_g0, sem_g1, sem_g2, sem_m0, sem_m1, sem_m2):
    core = lax.axis_index("c")
    sid = lax.axis_index("s")
    wid = sid * _NC + core
    bui = (bui0, bui1, bui2)
    bug = (bug0, bug1, bug2)
    bvi = (bvi0, bvi1, bvi2)
    msg = (msg0, msg1, msg2)
    sem_g = (sem_g0, sem_g1, sem_g2)
    sem_m = (sem_m0, sem_m1, sem_m2)

    def iv(ref, s):
        return ref[s // 8, pl.ds(pl.multiple_of((s % 8) * _C, _C), _C)]

    def fire(b, s):
        ivs = iv(idx_s, s)
        ivd = iv(idx_d, s)
        pltpu.async_copy(ui_hbm.at[ivs], bui[b], sem_g[b])
        pltpu.async_copy(ug_hbm.at[ivs], bug[b], sem_g[b])
        pltpu.async_copy(vi_hbm.at[ivd], bvi[b], sem_g[b])

    def wait_gathers(b, s):
        ivs = iv(idx_s, s)
        ivd = iv(idx_d, s)
        pltpu.make_async_copy(ui_hbm.at[ivs], bui[b], sem_g[b]).wait()
        pltpu.make_async_copy(ug_hbm.at[ivs], bug[b], sem_g[b]).wait()
        pltpu.make_async_copy(vi_hbm.at[ivd], bvi[b], sem_g[b]).wait()

    pltpu.sync_copy(g1w_hbm, p1w)
    pltpu.sync_copy(g1b_hbm, p1b)
    pltpu.sync_copy(g2w_hbm, p2w)
    pltpu.sync_copy(g2b_hbm, p2b)
    row0 = wid * (_STEPS // 8)
    pltpu.sync_copy(src_hbm.at[pl.ds(row0, _STEPS // 8)], idx_s)
    pltpu.sync_copy(dst_hbm.at[pl.ds(row0, _STEPS // 8)], idx_d)
    pltpu.sync_copy(z_hbm.at[pl.ds(sid * _RPS, _RPS)],
                    acc.at[pl.ds(sid * _RPS, _RPS)])
    plsc.subcore_barrier()

    p1wv = [p1w[pl.ds(16 * k, 16)] for k in range(_NNZ // 16)]
    p1bv = [p1b[pl.ds(16 * k, 16)] for k in range(_NNZ // 16)]
    p2wv = [p2w[pl.ds(16 * k, 16)] for k in range(_NNZ // 16)]
    p2bv = p2b[...]

    def compute(b):
        ui, ug, vi, mg = bui[b], bug[b], bvi[b], msg[b]

        @plsc.parallel_loop(0, _C, unroll=2)
        def edge(e):
            accs = [jnp.zeros((16,), _F32) for _ in range(4)]
            for k in range(_DP // 16):
                u = ui[e, pl.ds(16 * k, 16)]
                v = vi[e, pl.ds(16 * k, 16)]
                accs[k % 4] = accs[k % 4] + u * v
            inv = _lane_sum(accs[0] + accs[1] + accs[2] + accs[3])
            ga = p2bv
            for k in range(_NNZ // 16):
                tt = inv * p1wv[k] + p1bv[k]
                sg = 1.0 / (1.0 + jnp.exp(-tt))
                ga = ga + (tt * sg) * p2wv[k]
            gate = 1.0 / (1.0 + jnp.exp(-_lane_sum(ga)))
            for k in range(_NNZ // 16):
                mg[e, pl.ds(16 * k, 16)] = gate * ug[e, pl.ds(16 * k, 16)]

    def phase(b, s):
        wait_gathers(b, s)

        @pl.when(s >= 3)
        def _():
            pvd = iv(idx_d, s - 3)
            pltpu.make_async_copy(msg[b], acc.at[pvd], sem_m[b]).wait()

        compute(b)
        pltpu.async_copy(msg[b], acc.at[iv(idx_d, s)], sem_m[b], add=True)

        @pl.when(s + 3 < _STEPS)
        def _():
            fire(b, s + 3)

    for b in range(3):
        fire(b, b)

    def superstep(k, carry):
        for b in range(3):
            phase(b, 3 * k + b)
        return carry

    nfull = (_STEPS // 3) * 3
    lax.fori_loop(0, _STEPS // 3, superstep, 0)
    for s in range(nfull, _STEPS):
        phase(s % 3, jnp.int32(s))
    for s in range(_STEPS - 3, _STEPS):
        pltpu.make_async_copy(msg[s % 3], acc.at[iv(idx_d, s)],
                              sem_m[s % 3]).wait()
    plsc.subcore_barrier()
    pltpu.sync_copy(acc.at[pl.ds(sid * _RPS, _RPS)],
                    out_hbm.at[core, pl.ds(sid * _RPS, _RPS)])


def _edge_stage(Ui, Ug, Vi, src, dst, g1w, g1b, g2w, g2b16, zrows):
    call = pl.kernel(
        _edge_body,
        out_type=jax.ShapeDtypeStruct((_NC, _NP, _NNZ), _F32),
        mesh=plsc.VectorSubcoreMesh(core_axis_name="c", subcore_axis_name="s",
                                    num_cores=_NC, num_subcores=_NS),
        scratch_types=(
            [pltpu.VMEM((_STEPS // 8, 128), jnp.int32)] * 2
            + [pltpu.VMEM((_C, _DP), _F32)] * 3
            + [pltpu.VMEM((_C, _NNZ), _F32)] * 3
            + [pltpu.VMEM((_C, _DP), _F32)] * 3
            + [pltpu.VMEM((_C, _NNZ), _F32)] * 3
            + [pltpu.VMEM((_NNZ,), _F32)] * 3
            + [pltpu.VMEM((16,), _F32)]
            + [pltpu.VMEM_SHARED((_NP, _NNZ), _F32)]
            + [pltpu.SemaphoreType.DMA] * 6
        ),
    )
    return call(Ui, Ug, Vi, src, dst, g1w, g1b, g2w, g2b16, zrows)


def kernel(x, edge_index, sc_i, sc_j, sc_k, sc_c, W_in1, b_in1, W_in2, b_in2,
           K_mat, g1_w, g1_b, g2_w, g2_b, Wo1, bo1, Wo2, bo2):
    npad = _EPAD - _E
    src = jnp.concatenate([edge_index[0], jnp.zeros((npad,), jnp.int32)])
    dst = jnp.concatenate([edge_index[1],
                           jnp.full((npad,), _NP - 1, jnp.int32)])
    src = src.reshape(-1, 128)
    dst = dst.reshape(-1, 128)
    dr = jnp.arange(_D, dtype=jnp.int32)
    Si = (dr[:, None] == sc_i[None, :]).astype(_F32) * sc_c[None, :]
    Sj = (dr[:, None] == sc_j[None, :]).astype(_F32)
    Sk = (sc_k[:, None] == dr[None, :]).astype(_F32)
    pad = jnp.zeros((_D, _DP - _D), _F32)
    MUi = jnp.concatenate([K_mat, pad], axis=1)
    zrows = jnp.zeros((_NP, _NNZ), _F32)

    h = _mlp_in(x, W_in1, b_in1.reshape(1, -1), W_in2, b_in2.reshape(1, -1))
    for l in range(_L):
        Ui, Ug, Vi = _uv(h, MUi, Si)
        g2b16 = jnp.full((16,), g2_b[l] / 16.0, _F32)
        acc = _edge_stage(Ui, Ug, Vi, src, dst,
                          g1_w[l], g1_b[l], g2_w[l], g2b16, zrows)
        h = _update(h, acc[0, :_N], acc[1, :_N], Sj, Sk)
    return _readout(h, K_mat, Wo1, bo1.reshape(1, -1), Wo2, bo2.reshape(1, -1))
